# Initial kernel scaffold; baseline (speedup 1.0000x reference)
import jax
import jax.numpy as jnp
from jax.experimental import pallas as pl


def kernel(x, gate_w, W1, W2, W3, SW1, SW2, SW3):
    raise NotImplementedError("write your pallas kernel here")


# speedup vs baseline: 2.7519x; 2.7519x over previous
import jax
import jax.numpy as jnp
from jax import lax
from jax.experimental import pallas as pl
from jax.experimental.pallas import tpu as pltpu

T = 256
E = 64
TOP_K = 8
DIM = 1024
INTER = 256
S_INTER = 512


def _dot_t(a, b):
    return lax.dot_general(a, b, (((1,), (1,)), ((), ())),
                           preferred_element_type=jnp.float32)


def _moe_body(x_ref, gw_ref, w1_ref, w2_ref, w3_ref, sw1_ref, sw2_ref,
              sw3_ref, o_ref, wd_ref, xb_ref):
    i = pl.program_id(0)

    @pl.when(i == 0)
    def _gate_and_shared():
        xf = x_ref[...]
        xb = xf.astype(jnp.bfloat16)
        xb_ref[...] = xb

        logits = lax.dot_general(xf, gw_ref[...], (((1,), (1,)), ((), ())),
                                 preferred_element_type=jnp.float32)
        scores = jax.nn.sigmoid(logits)
        col = lax.broadcasted_iota(jnp.int32, (T, E), 1)
        masked = scores
        ssum = jnp.zeros((T, 1), jnp.float32)
        wd = jnp.zeros((T, E), jnp.float32)
        for _ in range(TOP_K):
            m = jnp.max(masked, axis=1, keepdims=True)
            eq = masked == m
            minidx = jnp.min(jnp.where(eq, col, E), axis=1, keepdims=True)
            first = col == minidx
            wd = wd + jnp.where(first, scores, 0.0)
            ssum = ssum + m
            masked = jnp.where(first, -1.0, masked)
        wd_ref[...] = wd / ssum

        s1 = _dot_t(xb, sw1_ref[...].astype(jnp.bfloat16))
        s3 = _dot_t(xb, sw3_ref[...].astype(jnp.bfloat16))
        hs = (s1 * jax.nn.sigmoid(s1) * s3).astype(jnp.bfloat16)
        o_ref[...] = _dot_t(hs, sw2_ref[...].astype(jnp.bfloat16))

    @pl.when(i > 0)
    def _expert():
        e = i - 1
        xb = xb_ref[...]
        h1 = _dot_t(xb, w1_ref[0].astype(jnp.bfloat16))
        h3 = _dot_t(xb, w3_ref[0].astype(jnp.bfloat16))
        h = (h1 * jax.nn.sigmoid(h1) * h3).astype(jnp.bfloat16)
        yc = _dot_t(h, w2_ref[0].astype(jnp.bfloat16))
        onehot = (lax.broadcasted_iota(jnp.int32, (E, 1), 0) == e
                  ).astype(jnp.float32)
        wcol = jnp.dot(wd_ref[...], onehot,
                       preferred_element_type=jnp.float32)
        o_ref[...] += yc * wcol


def kernel(x, gate_w, W1, W2, W3, SW1, SW2, SW3):
    grid = (E + 1,)
    const = lambda shape: pl.BlockSpec(shape, lambda i: (0,) * len(shape))
    ew = lambda shape: pl.BlockSpec(
        shape, lambda i: (jnp.maximum(i - 1, 0), 0, 0))
    return pl.pallas_call(
        _moe_body,
        grid=grid,
        in_specs=[
            const((T, DIM)),
            const((E, DIM)),
            ew((1, INTER, DIM)),
            ew((1, DIM, INTER)),
            ew((1, INTER, DIM)),
            const((S_INTER, DIM)),
            const((DIM, S_INTER)),
            const((S_INTER, DIM)),
        ],
        out_specs=const((T, DIM)),
        out_shape=jax.ShapeDtypeStruct((T, DIM), jnp.float32),
        scratch_shapes=[
            pltpu.VMEM((T, E), jnp.float32),
            pltpu.VMEM((T, DIM), jnp.bfloat16),
        ],
    )(x, gate_w, W1, W2, W3, SW1, SW2, SW3)


# speedup vs baseline: 2.7787x; 1.0097x over previous
import jax
import jax.numpy as jnp
from jax import lax
from jax.experimental import pallas as pl
from jax.experimental.pallas import tpu as pltpu

T = 256
E = 64
TOP_K = 8
DIM = 1024
INTER = 256
S_INTER = 512


def _dot_t(a, b):
    return lax.dot_general(a, b, (((1,), (1,)), ((), ())),
                           preferred_element_type=jnp.float32)


def _moe_body(x_ref, gw_ref, w1_ref, w2_ref, w3_ref, sw1_ref, sw2_ref,
              sw3_ref, o_ref, wd_ref):
    i = pl.program_id(0)

    @pl.when(i == 0)
    def _gate_and_shared():
        xf = x_ref[...]

        logits = _dot_t(xf, gw_ref[...])
        scores = jax.nn.sigmoid(logits)
        col = lax.broadcasted_iota(jnp.int32, (T, E), 1)
        masked = scores
        ssum = jnp.zeros((T, 1), jnp.float32)
        wd = jnp.zeros((T, E), jnp.float32)
        for _ in range(TOP_K):
            m = jnp.max(masked, axis=1, keepdims=True)
            eq = masked == m
            minidx = jnp.min(jnp.where(eq, col, E), axis=1, keepdims=True)
            first = col == minidx
            wd = wd + jnp.where(first, scores, 0.0)
            ssum = ssum + m
            masked = jnp.where(first, -1.0, masked)
        wd_ref[...] = wd / ssum

        s1 = _dot_t(xf, sw1_ref[...])
        s3 = _dot_t(xf, sw3_ref[...])
        hs = s1 * jax.nn.sigmoid(s1) * s3
        o_ref[...] = _dot_t(hs, sw2_ref[...])

    @pl.when(i > 0)
    def _expert():
        e = i - 1
        xf = x_ref[...]
        h1 = _dot_t(xf, w1_ref[0])
        h3 = _dot_t(xf, w3_ref[0])
        h = h1 * jax.nn.sigmoid(h1) * h3
        yc = _dot_t(h, w2_ref[0])
        onehot = (lax.broadcasted_iota(jnp.int32, (E, 1), 0) == e
                  ).astype(jnp.float32)
        wcol = jnp.dot(wd_ref[...], onehot,
                       preferred_element_type=jnp.float32)
        o_ref[...] += yc * wcol


def kernel(x, gate_w, W1, W2, W3, SW1, SW2, SW3):
    grid = (E + 1,)
    const = lambda shape: pl.BlockSpec(shape, lambda i: (0,) * len(shape))
    ew = lambda shape: pl.BlockSpec(
        shape, lambda i: (jnp.maximum(i - 1, 0), 0, 0))
    return pl.pallas_call(
        _moe_body,
        grid=grid,
        in_specs=[
            const((T, DIM)),
            const((E, DIM)),
            ew((1, INTER, DIM)),
            ew((1, DIM, INTER)),
            ew((1, INTER, DIM)),
            const((S_INTER, DIM)),
            const((DIM, S_INTER)),
            const((S_INTER, DIM)),
        ],
        out_specs=const((T, DIM)),
        out_shape=jax.ShapeDtypeStruct((T, DIM), jnp.float32),
        scratch_shapes=[
            pltpu.VMEM((T, E), jnp.float32),
        ],
    )(x, gate_w, W1, W2, W3, SW1, SW2, SW3)


# speedup vs baseline: 3.8149x; 1.3729x over previous
import jax
import jax.numpy as jnp
from jax import lax
from jax.experimental import pallas as pl
from jax.experimental.pallas import tpu as pltpu

T = 256
E = 64
TOP_K = 8
DIM = 1024
INTER = 256
S_INTER = 512
E_BLK = 4


def _dot_t(a, b):
    return lax.dot_general(a, b, (((1,), (1,)), ((), ())),
                           preferred_element_type=jnp.float32)


def _moe_body(x_ref, gw_ref, w1_ref, w2_ref, w3_ref, sw1_ref, sw2_ref,
              sw3_ref, o_ref, wd_ref):
    i = pl.program_id(0)

    @pl.when(i == 0)
    def _gate_and_shared():
        xf = x_ref[...]

        logits = _dot_t(xf, gw_ref[...])
        scores = jax.nn.sigmoid(logits)
        col = lax.broadcasted_iota(jnp.int32, (T, E), 1)
        masked = scores
        ssum = jnp.zeros((T, 1), jnp.float32)
        wd = jnp.zeros((T, E), jnp.float32)
        for _ in range(TOP_K):
            m = jnp.max(masked, axis=1, keepdims=True)
            eq = masked == m
            minidx = jnp.min(jnp.where(eq, col, E), axis=1, keepdims=True)
            first = col == minidx
            wd = wd + jnp.where(first, scores, 0.0)
            ssum = ssum + m
            masked = jnp.where(first, -1.0, masked)
        wd_ref[...] = wd / ssum

        s1 = _dot_t(xf, sw1_ref[...])
        s3 = _dot_t(xf, sw3_ref[...])
        hs = s1 * jax.nn.sigmoid(s1) * s3
        o_ref[...] = _dot_t(hs, sw2_ref[...])

    @pl.when(i > 0)
    def _expert():
        e0 = (i - 1) * E_BLK
        xf = x_ref[...]
        wd = wd_ref[...]
        eiota = lax.broadcasted_iota(jnp.int32, (E, 1), 0)
        acc = None
        for j in range(E_BLK):
            h1 = _dot_t(xf, w1_ref[j])
            h3 = _dot_t(xf, w3_ref[j])
            h = h1 * jax.nn.sigmoid(h1) * h3
            yc = _dot_t(h, w2_ref[j])
            onehot = (eiota == (e0 + j)).astype(jnp.float32)
            wcol = jnp.dot(wd, onehot, preferred_element_type=jnp.float32)
            contrib = yc * wcol
            acc = contrib if acc is None else acc + contrib
        o_ref[...] += acc


def kernel(x, gate_w, W1, W2, W3, SW1, SW2, SW3):
    grid = (E // E_BLK + 1,)
    const = lambda shape: pl.BlockSpec(shape, lambda i: (0,) * len(shape))
    ew = lambda shape: pl.BlockSpec(
        shape, lambda i: (jnp.maximum(i - 1, 0), 0, 0))
    return pl.pallas_call(
        _moe_body,
        grid=grid,
        in_specs=[
            const((T, DIM)),
            const((E, DIM)),
            ew((E_BLK, INTER, DIM)),
            ew((E_BLK, DIM, INTER)),
            ew((E_BLK, INTER, DIM)),
            const((S_INTER, DIM)),
            const((DIM, S_INTER)),
            const((S_INTER, DIM)),
        ],
        out_specs=const((T, DIM)),
        out_shape=jax.ShapeDtypeStruct((T, DIM), jnp.float32),
        scratch_shapes=[
            pltpu.VMEM((T, E), jnp.float32),
        ],
    )(x, gate_w, W1, W2, W3, SW1, SW2, SW3)
